# baseline (speedup 1.0000x reference)
import functools

import jax
import jax.numpy as jnp
from jax import lax
from jax.experimental import pallas as pl
from jax.experimental.pallas import tpu as pltpu
from jax.experimental.pallas import tpu_sc as plsc

EXT = 500
STRIP = 1664
SPAD = 512
NV = 50000
VX = NV + EXT
NFULL = 30
TAILV = NV - NFULL * STRIP
TAILW = 640


def _sc_body(vp_ref, vpt_ref, art_ref, awt_ref, gen_ref, attn_ref, out_ref,
             vps, awt, idx, genv, attnv, vptail, *, bsz, n_agents):
    c = lax.axis_index("c")
    s = lax.axis_index("s")
    wid = c * 16 + s
    base = wid * STRIP
    nj = n_agents * SPAD // 16

    pltpu.sync_copy(art_ref, idx)
    pltpu.sync_copy(gen_ref, genv)
    pltpu.sync_copy(attn_ref, attnv)

    def body_b(b, _):
        pltpu.sync_copy(awt_ref.at[b], awt)

        @pl.when(wid < NFULL)
        def _full_load():
            pltpu.sync_copy(vp_ref.at[b, :, pl.ds(base, STRIP)], vps)

        @pl.when(wid == NFULL)
        def _tail_load():
            def zrow(q, _):
                zv = jnp.zeros((16,), jnp.float32)
                for t in range(32):
                    vps[t, pl.ds(q * 16, 16)] = zv
                return 0
            lax.fori_loop(0, STRIP // 16, zrow, 0)
            pltpu.sync_copy(vpt_ref.at[b], vptail)
            for t in range(32):
                for q in range(TAILV // 16):
                    vps[t, pl.ds(q * 16, 16)] = vptail[t, pl.ds(q * 16, 16)]

        w0 = jnp.zeros((16,), jnp.float32)
        w1 = jnp.zeros((16,), jnp.float32)
        for a in range(n_agents):
            w0 = w0 + (attnv[pl.ds(b * 96 + a * 32, 16)]
                       * genv[pl.ds(b * 96 + a * 32, 16)])
            w1 = w1 + (attnv[pl.ds(b * 96 + a * 32 + 16, 16)]
                       * genv[pl.ds(b * 96 + a * 32 + 16, 16)])
        wts = [w0[t] for t in range(16)] + [w1[t] for t in range(16)]

        def scale(qq, _):
            for t in range(32):
                vps[t, pl.ds(qq * 16, 16)] = (
                    vps[t, pl.ds(qq * 16, 16)]
                    * jnp.full((16,), wts[t], jnp.float32))
            return 0
        lax.fori_loop(0, STRIP // 16, scale, 0)

        def jvec(q, _):
            jv = idx[pl.ds(b * n_agents * SPAD + q * 16, 16)]
            lv = jv - base
            a = q // (SPAD // 16)
            k0 = (attnv[pl.ds(b * 96 + a * 32, 16)]
                  * (1.0 - genv[pl.ds(b * 96 + a * 32, 16)]))
            k1 = (attnv[pl.ds(b * 96 + a * 32 + 16, 16)]
                  * (1.0 - genv[pl.ds(b * 96 + a * 32 + 16, 16)]))
            for l in range(16):
                ll = lv[l]

                @pl.when((ll >= 0) & (ll < STRIP))
                def _one(l=l):
                    row = q * 4 + l // 4
                    off = (l % 4) * 32
                    v0 = awt[row, pl.ds(off, 16)] * k0
                    v1 = awt[row, pl.ds(off + 16, 16)] * k1
                    cb = (ll // 16) * 16
                    d = (jax.lax.broadcasted_iota(jnp.int32, (16,), 0)
                         - jnp.full((16,), ll % 16, jnp.int32))
                    oh = (1 - jnp.minimum(jnp.abs(d), 1)).astype(jnp.float32)
                    for t in range(32):
                        vt = v0[t] if t < 16 else v1[t - 16]
                        vps[t, pl.ds(cb, 16)] = (
                            vps[t, pl.ds(cb, 16)]
                            + jnp.full((16,), vt, jnp.float32) * oh)
            return 0
        lax.fori_loop(0, nj, jvec, 0)

        pltpu.sync_copy(vps, out_ref.at[b, :, pl.ds(base, STRIP)])
        return 0
    lax.fori_loop(0, bsz, body_b, 0)


def kernel(article, vocab_probs, generation_probs, agentwise_attn, agent_attn):
    bsz, n_agents, src_len = article.shape
    tgt_len, n_vocab = vocab_probs.shape[1], vocab_probs.shape[2]

    art_p = jnp.pad(article.astype(jnp.int32),
                    ((0, 0), (0, 0), (0, SPAD - src_len)))
    art_flat = art_p.reshape(bsz * n_agents * SPAD)
    aw_p = jnp.pad(agentwise_attn, ((0, 0), (0, 0), (0, 0),
                                    (0, SPAD - src_len)))
    awt_h = aw_p.transpose(0, 2, 3, 1).reshape(
        bsz, n_agents * SPAD // 4, 128)
    gen_flat = generation_probs.transpose(0, 2, 1).reshape(-1)
    attn_flat = agent_attn.transpose(0, 2, 1).reshape(-1)
    vp_tail = vocab_probs[:, :, NFULL * STRIP:]

    mesh = plsc.VectorSubcoreMesh(core_axis_name="c", subcore_axis_name="s",
                                  num_cores=2, num_subcores=16)
    body = functools.partial(_sc_body, bsz=bsz, n_agents=n_agents)
    f = pl.kernel(
        body,
        out_type=jax.ShapeDtypeStruct((bsz, tgt_len, 32 * STRIP), jnp.float32),
        mesh=mesh,
        scratch_types=[
            pltpu.VMEM((32, STRIP), jnp.float32),
            pltpu.VMEM((n_agents * SPAD // 4, 128), jnp.float32),
            pltpu.VMEM((bsz * n_agents * SPAD,), jnp.int32),
            pltpu.VMEM((bsz * n_agents * 32,), jnp.float32),
            pltpu.VMEM((bsz * n_agents * 32,), jnp.float32),
            pltpu.VMEM((tgt_len, TAILV), jnp.float32),
        ],
    )
    out = f(vocab_probs, vp_tail, art_flat, awt_h, gen_flat, attn_flat)
    return out[:, :, :VX]


# speedup vs baseline: 1.5443x; 1.5443x over previous
import functools

import jax
import jax.numpy as jnp
from jax import lax
from jax.experimental import pallas as pl
from jax.experimental.pallas import tpu as pltpu
from jax.experimental.pallas import tpu_sc as plsc

EXT = 500
STRIP = 1664
SPAD = 512
NV = 50000
VX = NV + EXT
NFULL = 30
TAILV = NV - NFULL * STRIP
TAILW = 640


def _sc_body(vp_ref, vpt_ref, art_ref, awt_ref, gen_ref, attn_ref, out_ref,
             vps, awt, idx, genv, attnv, vptail, hitq, *, bsz, n_agents):
    c = lax.axis_index("c")
    s = lax.axis_index("s")
    wid = c * 16 + s
    base = wid * STRIP
    nj = n_agents * SPAD // 16

    pltpu.sync_copy(art_ref, idx)
    pltpu.sync_copy(gen_ref, genv)
    pltpu.sync_copy(attn_ref, attnv)

    def body_b(b, _):
        pltpu.sync_copy(awt_ref.at[b], awt)

        @pl.when(wid < NFULL)
        def _full_load():
            pltpu.sync_copy(vp_ref.at[b, :, pl.ds(base, STRIP)], vps)

        @pl.when(wid == NFULL)
        def _tail_load():
            def zrow(q, _):
                zv = jnp.zeros((16,), jnp.float32)
                for t in range(32):
                    vps[t, pl.ds(q * 16, 16)] = zv
                return 0
            lax.fori_loop(0, STRIP // 16, zrow, 0)
            pltpu.sync_copy(vpt_ref.at[b], vptail)
            for t in range(32):
                for q in range(TAILV // 16):
                    vps[t, pl.ds(q * 16, 16)] = vptail[t, pl.ds(q * 16, 16)]

        w0 = jnp.zeros((16,), jnp.float32)
        w1 = jnp.zeros((16,), jnp.float32)
        for a in range(n_agents):
            w0 = w0 + (attnv[pl.ds(b * 96 + a * 32, 16)]
                       * genv[pl.ds(b * 96 + a * 32, 16)])
            w1 = w1 + (attnv[pl.ds(b * 96 + a * 32 + 16, 16)]
                       * genv[pl.ds(b * 96 + a * 32 + 16, 16)])
        wts = [w0[t] for t in range(16)] + [w1[t] for t in range(16)]

        def scale(qq, _):
            for t in range(32):
                vps[t, pl.ds(qq * 16, 16)] = (
                    vps[t, pl.ds(qq * 16, 16)]
                    * jnp.full((16,), wts[t], jnp.float32))
            return 0
        lax.fori_loop(0, STRIP // 16, scale, 0)

        def jvec(g, cnt):
            jb = b * n_agents * SPAD + g * 64
            lvs = []
            hs = []
            for k4 in range(4):
                lv = idx[pl.ds(jb + k4 * 16, 16)] - base
                cl = jnp.minimum(jnp.maximum(lv, 0), STRIP - 1)
                hs.append(1 - jnp.minimum(jnp.abs(lv - cl), 1))
                lvs.append(lv)
            wv = hs[0] + 2 * hs[1] + 4 * hs[2] + 8 * hs[3]

            for wl in range(16):
                cw = wv[wl]

                def record(c, wl=wl):
                    for k4 in range(4):
                        ll = lvs[k4][wl]
                        hit = hs[k4][wl]
                        j_local = g * 64 + k4 * 16 + wl
                        hitq[c] = ll + j_local * 2048
                        c = c + hit
                    return c
                cnt = lax.cond(cw != 0, record, lambda cc: cc, cnt)
            return cnt
        nhits = lax.fori_loop(0, nj // 4, jvec, 0)

        def apply_hit(i, _):
            e = hitq[i]
            ll = lax.rem(e, 2048)
            j = lax.div(e, 2048)
            a = lax.div(j, SPAD)
            row = lax.div(j, 4)
            off = lax.rem(j, 4) * 32
            k0 = (attnv[pl.ds(b * 96 + a * 32, 16)]
                  * (1.0 - genv[pl.ds(b * 96 + a * 32, 16)]))
            k1 = (attnv[pl.ds(b * 96 + a * 32 + 16, 16)]
                  * (1.0 - genv[pl.ds(b * 96 + a * 32 + 16, 16)]))
            v0 = awt[row, pl.ds(off, 16)] * k0
            v1 = awt[row, pl.ds(off + 16, 16)] * k1
            cb = lax.div(ll, 16) * 16
            d = (jax.lax.broadcasted_iota(jnp.int32, (16,), 0)
                 - jnp.full((16,), lax.rem(ll, 16), jnp.int32))
            oh = (1 - jnp.minimum(jnp.abs(d), 1)).astype(jnp.float32)
            for t in range(32):
                vt = v0[t] if t < 16 else v1[t - 16]
                vps[t, pl.ds(cb, 16)] = (
                    vps[t, pl.ds(cb, 16)]
                    + jnp.full((16,), vt, jnp.float32) * oh)
            return 0
        lax.fori_loop(0, nhits, apply_hit, 0)

        pltpu.sync_copy(vps, out_ref.at[b, :, pl.ds(base, STRIP)])
        return 0
    lax.fori_loop(0, bsz, body_b, 0)


def kernel(article, vocab_probs, generation_probs, agentwise_attn, agent_attn):
    bsz, n_agents, src_len = article.shape
    tgt_len, n_vocab = vocab_probs.shape[1], vocab_probs.shape[2]

    art_p = jnp.pad(article.astype(jnp.int32),
                    ((0, 0), (0, 0), (0, SPAD - src_len)),
                    constant_values=-1)
    art_flat = art_p.reshape(bsz * n_agents * SPAD)
    aw_p = jnp.pad(agentwise_attn, ((0, 0), (0, 0), (0, 0),
                                    (0, SPAD - src_len)))
    awt_h = aw_p.transpose(0, 2, 3, 1).reshape(
        bsz, n_agents * SPAD // 4, 128)
    gen_flat = generation_probs.transpose(0, 2, 1).reshape(-1)
    attn_flat = agent_attn.transpose(0, 2, 1).reshape(-1)
    vp_tail = vocab_probs[:, :, NFULL * STRIP:]

    mesh = plsc.VectorSubcoreMesh(core_axis_name="c", subcore_axis_name="s",
                                  num_cores=2, num_subcores=16)
    body = functools.partial(_sc_body, bsz=bsz, n_agents=n_agents)
    f = pl.kernel(
        body,
        out_type=jax.ShapeDtypeStruct((bsz, tgt_len, 32 * STRIP), jnp.float32),
        mesh=mesh,
        scratch_types=[
            pltpu.VMEM((32, STRIP), jnp.float32),
            pltpu.VMEM((n_agents * SPAD // 4, 128), jnp.float32),
            pltpu.VMEM((bsz * n_agents * SPAD,), jnp.int32),
            pltpu.VMEM((bsz * n_agents * 32,), jnp.float32),
            pltpu.VMEM((bsz * n_agents * 32,), jnp.float32),
            pltpu.VMEM((tgt_len, TAILV), jnp.float32),
            pltpu.SMEM((n_agents * src_len + 1,), jnp.int32),
        ],
    )
    out = f(vocab_probs, vp_tail, art_flat, awt_h, gen_flat, attn_flat)
    return out[:, :, :VX]


# speedup vs baseline: 1.8885x; 1.2229x over previous
import functools

import jax
import jax.numpy as jnp
from jax import lax
from jax.experimental import pallas as pl
from jax.experimental.pallas import tpu as pltpu
from jax.experimental.pallas import tpu_sc as plsc

EXT = 500
STRIP = 1664
SPAD = 512
NV = 50000
VX = NV + EXT
NFULL = 30
TAILV = NV - NFULL * STRIP
TAILW = 640


def _sc_body(vp_ref, vpt_ref, art_ref, awt_ref, gen_ref, attn_ref, out_ref,
             vps, awt, idx, genv, attnv, vptail, hitq, asem, vsem, *, bsz, n_agents):
    c = lax.axis_index("c")
    s = lax.axis_index("s")
    wid = c * 16 + s
    base = wid * STRIP
    nj = n_agents * SPAD // 16

    pltpu.sync_copy(art_ref, idx)
    pltpu.sync_copy(gen_ref, genv)
    pltpu.sync_copy(attn_ref, attnv)

    lbase = jnp.minimum(base, (NFULL - 1) * STRIP)

    def body_b(b, _):
        awt_cp = pltpu.async_copy(awt_ref.at[b], awt, asem)
        vp_cp = pltpu.async_copy(
            vp_ref.at[b, :, pl.ds(lbase, STRIP)], vps, vsem)

        w0 = jnp.zeros((16,), jnp.float32)
        w1 = jnp.zeros((16,), jnp.float32)
        for a in range(n_agents):
            w0 = w0 + (attnv[pl.ds(b * 96 + a * 32, 16)]
                       * genv[pl.ds(b * 96 + a * 32, 16)])
            w1 = w1 + (attnv[pl.ds(b * 96 + a * 32 + 16, 16)]
                       * genv[pl.ds(b * 96 + a * 32 + 16, 16)])
        wts = [w0[t] for t in range(16)] + [w1[t] for t in range(16)]

        def jvec(g, cnt):
            jb = b * n_agents * SPAD + g * 64
            lvs = []
            hs = []
            for k4 in range(4):
                lv = idx[pl.ds(jb + k4 * 16, 16)] - base
                cl = jnp.minimum(jnp.maximum(lv, 0), STRIP - 1)
                hs.append(1 - jnp.minimum(jnp.abs(lv - cl), 1))
                lvs.append(lv)
            wv = hs[0] + 2 * hs[1] + 4 * hs[2] + 8 * hs[3]

            for wl in range(16):
                cw = wv[wl]

                def record(c, wl=wl):
                    for k4 in range(4):
                        ll = lvs[k4][wl]
                        hit = hs[k4][wl]
                        j_local = g * 64 + k4 * 16 + wl
                        hitq[c] = ll + j_local * 2048
                        c = c + hit
                    return c
                cnt = lax.cond(cw != 0, record, lambda cc: cc, cnt)
            return cnt
        nhits = lax.fori_loop(0, nj // 4, jvec, 0)

        vp_cp.wait()

        @pl.when(wid == NFULL)
        def _tail_load():
            def zrow(q, _):
                zv = jnp.zeros((16,), jnp.float32)
                for t in range(32):
                    vps[t, pl.ds(q * 16, 16)] = zv
                return 0
            lax.fori_loop(0, STRIP // 16, zrow, 0)
            pltpu.sync_copy(vpt_ref.at[b], vptail)
            for t in range(32):
                for q in range(TAILV // 16):
                    vps[t, pl.ds(q * 16, 16)] = vptail[t, pl.ds(q * 16, 16)]

        def scale(qq, _):
            for t in range(32):
                vps[t, pl.ds(qq * 16, 16)] = (
                    vps[t, pl.ds(qq * 16, 16)]
                    * jnp.full((16,), wts[t], jnp.float32))
            return 0
        lax.fori_loop(0, STRIP // 16, scale, 0)

        awt_cp.wait()

        def apply_hit(i, _):
            e = hitq[i]
            ll = e & 2047
            j = lax.shift_right_logical(e, 11)
            a = lax.shift_right_logical(j, 9)
            row = lax.shift_right_logical(j, 2)
            off = (j & 3) * 32
            k0 = (attnv[pl.ds(b * 96 + a * 32, 16)]
                  * (1.0 - genv[pl.ds(b * 96 + a * 32, 16)]))
            k1 = (attnv[pl.ds(b * 96 + a * 32 + 16, 16)]
                  * (1.0 - genv[pl.ds(b * 96 + a * 32 + 16, 16)]))
            v0 = awt[row, pl.ds(off, 16)] * k0
            v1 = awt[row, pl.ds(off + 16, 16)] * k1
            cb = pl.multiple_of(ll & ~15, 16)
            d = (jax.lax.broadcasted_iota(jnp.int32, (16,), 0)
                 - jnp.full((16,), ll & 15, jnp.int32))
            oh = (1 - jnp.minimum(jnp.abs(d), 1)).astype(jnp.float32)
            for t in range(32):
                vt = v0[t] if t < 16 else v1[t - 16]
                vps[t, pl.ds(cb, 16)] = (
                    vps[t, pl.ds(cb, 16)]
                    + jnp.full((16,), vt, jnp.float32) * oh)
            return 0
        lax.fori_loop(0, nhits, apply_hit, 0)

        pltpu.sync_copy(vps, out_ref.at[b, :, pl.ds(base, STRIP)])
        return 0
    lax.fori_loop(0, bsz, body_b, 0)


def kernel(article, vocab_probs, generation_probs, agentwise_attn, agent_attn):
    bsz, n_agents, src_len = article.shape
    tgt_len, n_vocab = vocab_probs.shape[1], vocab_probs.shape[2]

    art_p = jnp.pad(article.astype(jnp.int32),
                    ((0, 0), (0, 0), (0, SPAD - src_len)),
                    constant_values=-1)
    art_flat = art_p.reshape(bsz * n_agents * SPAD)
    aw_p = jnp.pad(agentwise_attn, ((0, 0), (0, 0), (0, 0),
                                    (0, SPAD - src_len)))
    awt_h = aw_p.transpose(0, 2, 3, 1).reshape(
        bsz, n_agents * SPAD // 4, 128)
    gen_flat = generation_probs.transpose(0, 2, 1).reshape(-1)
    attn_flat = agent_attn.transpose(0, 2, 1).reshape(-1)
    vp_tail = vocab_probs[:, :, NFULL * STRIP:]

    mesh = plsc.VectorSubcoreMesh(core_axis_name="c", subcore_axis_name="s",
                                  num_cores=2, num_subcores=16)
    body = functools.partial(_sc_body, bsz=bsz, n_agents=n_agents)
    f = pl.kernel(
        body,
        out_type=jax.ShapeDtypeStruct((bsz, tgt_len, 32 * STRIP), jnp.float32),
        mesh=mesh,
        scratch_types=[
            pltpu.VMEM((32, STRIP), jnp.float32),
            pltpu.VMEM((n_agents * SPAD // 4, 128), jnp.float32),
            pltpu.VMEM((bsz * n_agents * SPAD,), jnp.int32),
            pltpu.VMEM((bsz * n_agents * 32,), jnp.float32),
            pltpu.VMEM((bsz * n_agents * 32,), jnp.float32),
            pltpu.VMEM((tgt_len, TAILV), jnp.float32),
            pltpu.SMEM((n_agents * src_len + 1,), jnp.int32),
            pltpu.SemaphoreType.DMA,
            pltpu.SemaphoreType.DMA,
        ],
    )
    out = f(vocab_probs, vp_tail, art_flat, awt_h, gen_flat, attn_flat)
    return out[:, :, :VX]


# speedup vs baseline: 1.9935x; 1.0556x over previous
import functools

import jax
import jax.numpy as jnp
from jax import lax
from jax.experimental import pallas as pl
from jax.experimental.pallas import tpu as pltpu
from jax.experimental.pallas import tpu_sc as plsc

EXT = 500
STRIP = 1664
SPAD = 512
NV = 50000
VX = NV + EXT
NFULL = 30
TAILV = NV - NFULL * STRIP
TAILW = 640


def _sc_body(vp_ref, vpt_ref, art_ref, awt_ref, gen_ref, attn_ref, out_ref,
             vps, awt, idx, genv, attnv, vptail, hitq, asem, vsem, *, bsz, n_agents):
    c = lax.axis_index("c")
    s = lax.axis_index("s")
    wid = c * 16 + s
    base = wid * STRIP
    nj = n_agents * SPAD // 16

    pltpu.sync_copy(art_ref, idx)
    pltpu.sync_copy(gen_ref, genv)
    pltpu.sync_copy(attn_ref, attnv)

    lbase = jnp.minimum(base, (NFULL - 1) * STRIP)

    def body_b(b, _):
        awt_cp = pltpu.async_copy(awt_ref.at[b], awt, asem)
        vp_cp = pltpu.async_copy(
            vp_ref.at[b, :, pl.ds(lbase, STRIP)], vps, vsem)

        w0 = jnp.zeros((16,), jnp.float32)
        w1 = jnp.zeros((16,), jnp.float32)
        for a in range(n_agents):
            w0 = w0 + (attnv[pl.ds(b * 96 + a * 32, 16)]
                       * genv[pl.ds(b * 96 + a * 32, 16)])
            w1 = w1 + (attnv[pl.ds(b * 96 + a * 32 + 16, 16)]
                       * genv[pl.ds(b * 96 + a * 32 + 16, 16)])
        wts = [w0[t] for t in range(16)] + [w1[t] for t in range(16)]

        def jvec(g, cnt):
            jb = b * n_agents * SPAD + g * 64
            lvs = []
            hs = []
            for k4 in range(4):
                lv = idx[pl.ds(jb + k4 * 16, 16)] - base
                cl = jnp.minimum(jnp.maximum(lv, 0), STRIP - 1)
                hs.append(1 - jnp.minimum(jnp.abs(lv - cl), 1))
                lvs.append(lv)
            wv = hs[0] + 2 * hs[1] + 4 * hs[2] + 8 * hs[3]

            for wl in range(16):
                cw = wv[wl]

                def record(c, wl=wl):
                    for k4 in range(4):
                        ll = lvs[k4][wl]
                        hit = hs[k4][wl]
                        j_local = g * 64 + k4 * 16 + wl
                        hitq[c] = ll + j_local * 2048
                        c = c + hit
                    return c
                cnt = lax.cond(cw != 0, record, lambda cc: cc, cnt)
            return cnt
        nhits = lax.fori_loop(0, nj // 4, jvec, 0)

        vp_cp.wait()

        @pl.when(wid == NFULL)
        def _tail_load():
            def zrow(q, _):
                zv = jnp.zeros((16,), jnp.float32)
                for t in range(32):
                    vps[t, pl.ds(q * 16, 16)] = zv
                return 0
            lax.fori_loop(0, 37, zrow, 0)
            pltpu.sync_copy(vpt_ref.at[b], vptail)
            for t in range(32):
                for q in range(TAILV // 16):
                    vps[t, pl.ds(q * 16, 16)] = vptail[t, pl.ds(q * 16, 16)]

        def scale(qq, _):
            for t in range(32):
                vps[t, pl.ds(qq * 16, 16)] = (
                    vps[t, pl.ds(qq * 16, 16)]
                    * jnp.full((16,), wts[t], jnp.float32))
            return 0
        lax.fori_loop(0, STRIP // 16, scale, 0)

        awt_cp.wait()

        def apply_hit(i, _):
            e = hitq[i]
            ll = e & 2047
            j = lax.shift_right_logical(e, 11)
            a = lax.shift_right_logical(j, 9)
            row = lax.shift_right_logical(j, 2)
            off = (j & 3) * 32
            k0 = (attnv[pl.ds(b * 96 + a * 32, 16)]
                  * (1.0 - genv[pl.ds(b * 96 + a * 32, 16)]))
            k1 = (attnv[pl.ds(b * 96 + a * 32 + 16, 16)]
                  * (1.0 - genv[pl.ds(b * 96 + a * 32 + 16, 16)]))
            v0 = awt[row, pl.ds(off, 16)] * k0
            v1 = awt[row, pl.ds(off + 16, 16)] * k1
            cb = pl.multiple_of(ll & ~15, 16)
            d = (jax.lax.broadcasted_iota(jnp.int32, (16,), 0)
                 - jnp.full((16,), ll & 15, jnp.int32))
            oh = (1 - jnp.minimum(jnp.abs(d), 1)).astype(jnp.float32)
            for t in range(32):
                vt = v0[t] if t < 16 else v1[t - 16]
                vps[t, pl.ds(cb, 16)] = (
                    vps[t, pl.ds(cb, 16)]
                    + jnp.full((16,), vt, jnp.float32) * oh)
            return 0
        lax.fori_loop(0, nhits, apply_hit, 0)

        pltpu.sync_copy(vps, out_ref.at[b, :, pl.ds(base, STRIP)])
        return 0
    lax.fori_loop(0, bsz, body_b, 0)


def kernel(article, vocab_probs, generation_probs, agentwise_attn, agent_attn):
    bsz, n_agents, src_len = article.shape
    tgt_len, n_vocab = vocab_probs.shape[1], vocab_probs.shape[2]

    art_p = jnp.pad(article.astype(jnp.int32),
                    ((0, 0), (0, 0), (0, SPAD - src_len)),
                    constant_values=-1)
    art_flat = art_p.reshape(bsz * n_agents * SPAD)
    aw_p = jnp.pad(agentwise_attn, ((0, 0), (0, 0), (0, 0),
                                    (0, SPAD - src_len)))
    awt_h = aw_p.transpose(0, 2, 3, 1).reshape(
        bsz, n_agents * SPAD // 4, 128)
    gen_flat = generation_probs.transpose(0, 2, 1).reshape(-1)
    attn_flat = agent_attn.transpose(0, 2, 1).reshape(-1)
    vp_tail = vocab_probs[:, :, NFULL * STRIP:]

    mesh = plsc.VectorSubcoreMesh(core_axis_name="c", subcore_axis_name="s",
                                  num_cores=2, num_subcores=16)
    body = functools.partial(_sc_body, bsz=bsz, n_agents=n_agents)
    f = pl.kernel(
        body,
        out_type=jax.ShapeDtypeStruct((bsz, tgt_len, 32 * STRIP), jnp.float32),
        mesh=mesh,
        scratch_types=[
            pltpu.VMEM((32, STRIP), jnp.float32),
            pltpu.VMEM((n_agents * SPAD // 4, 128), jnp.float32),
            pltpu.VMEM((bsz * n_agents * SPAD,), jnp.int32),
            pltpu.VMEM((bsz * n_agents * 32,), jnp.float32),
            pltpu.VMEM((bsz * n_agents * 32,), jnp.float32),
            pltpu.VMEM((tgt_len, TAILV), jnp.float32),
            pltpu.SMEM((n_agents * src_len + 1,), jnp.int32),
            pltpu.SemaphoreType.DMA,
            pltpu.SemaphoreType.DMA,
        ],
    )
    out = f(vocab_probs, vp_tail, art_flat, awt_h, gen_flat, attn_flat)
    return out[:, :, :VX]
